# baseline (speedup 1.0000x reference)
import jax
import jax.numpy as jnp
import numpy as np
from jax.experimental import pallas as pl

_N = 50000
_BN = 1000


def _mm_body(x_ref, w_ref, o_ref):
    o_ref[...] = jnp.dot(x_ref[...], w_ref[...],
                         preferred_element_type=jnp.float32)


def _mm(x, w):
    n, k = x.shape
    m = w.shape[1]
    return pl.pallas_call(
        _mm_body,
        grid=(n // _BN,),
        in_specs=[
            pl.BlockSpec((_BN, k), lambda i: (i, 0)),
            pl.BlockSpec((k, m), lambda i: (0, 0)),
        ],
        out_specs=pl.BlockSpec((_BN, m), lambda i: (i, 0)),
        out_shape=jax.ShapeDtypeStruct((n, m), jnp.float32),
    )(x, w)


def _gatpre_body(h_ref, w_ref, a_ref, hw_ref, al_ref):
    hw = jnp.dot(h_ref[...], w_ref[...], preferred_element_type=jnp.float32)
    hw_ref[...] = hw
    al_ref[...] = jnp.dot(hw, a_ref[...], preferred_element_type=jnp.float32)


def _gatpre(h, w, amat):
    n, k = h.shape
    m = w.shape[1]
    p = amat.shape[1]
    return pl.pallas_call(
        _gatpre_body,
        grid=(n // _BN,),
        in_specs=[
            pl.BlockSpec((_BN, k), lambda i: (i, 0)),
            pl.BlockSpec((k, m), lambda i: (0, 0)),
            pl.BlockSpec((m, p), lambda i: (0, 0)),
        ],
        out_specs=[
            pl.BlockSpec((_BN, m), lambda i: (i, 0)),
            pl.BlockSpec((_BN, p), lambda i: (i, 0)),
        ],
        out_shape=[
            jax.ShapeDtypeStruct((n, m), jnp.float32),
            jax.ShapeDtypeStruct((n, p), jnp.float32),
        ],
    )(h, w, amat)


def _attn_mat(a_src, a_dst):
    H, C = a_src.shape
    hid = H * C
    m = np.zeros((hid, 2 * H), dtype=np.float32)
    blk = np.zeros((hid, 2 * H), dtype=np.float32)
    rows = np.arange(hid)
    heads = rows // C
    m[rows, heads] = 1.0
    blk[rows, H + heads] = 1.0
    a_s_flat = jnp.reshape(a_src, (hid,))
    a_d_flat = jnp.reshape(a_dst, (hid,))
    return jnp.asarray(m) * a_s_flat[:, None] + jnp.asarray(blk) * a_d_flat[:, None]


def kernel(x, edge_index, W_gcn, b_gcn, W_gat0, a_src0, a_dst0, b_gat0,
           W_gat1, a_src1, a_dst1, b_gat1, W_gat2, a_src2, a_dst2, b_gat2,
           W_out, b_out):
    n = x.shape[0]
    H, C = a_src0.shape
    src = jnp.asarray(edge_index[0], jnp.int32)
    dst = jnp.asarray(edge_index[1], jnp.int32)
    loop = jnp.arange(n, dtype=jnp.int32)
    s = jnp.concatenate([src, loop])
    d = jnp.concatenate([dst, loop])

    hlin = _mm(x, W_gcn)
    deg = jax.ops.segment_sum(jnp.ones_like(d, dtype=jnp.float32), d,
                              num_segments=n)
    dinv = 1.0 / jnp.sqrt(jnp.clip(deg, 1.0))
    norm = dinv[s] * dinv[d]
    h = jax.ops.segment_sum(norm[:, None] * hlin[s], d, num_segments=n)
    h = jax.nn.relu(h + b_gcn)

    gats = [(W_gat0, a_src0, a_dst0, b_gat0),
            (W_gat1, a_src1, a_dst1, b_gat1),
            (W_gat2, a_src2, a_dst2, b_gat2)]
    for (Wg, a_s, a_d, bg) in gats:
        hw, al = _gatpre(h, Wg, _attn_mat(a_s, a_d))
        al_s = al[:, :H]
        al_d = al[:, H:]
        e = jax.nn.leaky_relu(al_s[s] + al_d[d], negative_slope=0.2)
        emax = jax.ops.segment_max(e, d, num_segments=n)
        emax = jnp.where(jnp.isfinite(emax), emax, 0.0)
        ee = jnp.exp(e - emax[d])
        denom = jax.ops.segment_sum(ee, d, num_segments=n)
        alpha = ee / jnp.clip(denom[d], 1e-16)
        hh = hw.reshape(n, H, C)
        h = jax.ops.segment_sum(alpha[:, :, None] * hh[s], d,
                                num_segments=n).reshape(n, H * C) + bg

    return _mm(h, W_out) + b_out


# speedup vs baseline: 4.3502x; 4.3502x over previous
import jax
import jax.numpy as jnp
import numpy as np
from jax.experimental import pallas as pl

_N = 50000
_BN = 1000


def _mm_body(x_ref, w_ref, o_ref):
    o_ref[...] = jnp.dot(x_ref[...], w_ref[...],
                         preferred_element_type=jnp.float32)


def _mm(x, w):
    n, k = x.shape
    m = w.shape[1]
    return pl.pallas_call(
        _mm_body,
        grid=(n // _BN,),
        in_specs=[
            pl.BlockSpec((_BN, k), lambda i: (i, 0)),
            pl.BlockSpec((k, m), lambda i: (0, 0)),
        ],
        out_specs=pl.BlockSpec((_BN, m), lambda i: (i, 0)),
        out_shape=jax.ShapeDtypeStruct((n, m), jnp.float32),
    )(x, w)


def _gatpre_body(h_ref, w_ref, a_ref, hw_ref, al_ref):
    hw = jnp.dot(h_ref[...], w_ref[...], preferred_element_type=jnp.float32)
    hw_ref[...] = hw
    al_ref[...] = jnp.dot(hw, a_ref[...], preferred_element_type=jnp.float32)


def _gatpre(h, w, amat):
    n, k = h.shape
    m = w.shape[1]
    p = amat.shape[1]
    return pl.pallas_call(
        _gatpre_body,
        grid=(n // _BN,),
        in_specs=[
            pl.BlockSpec((_BN, k), lambda i: (i, 0)),
            pl.BlockSpec((k, m), lambda i: (0, 0)),
            pl.BlockSpec((m, p), lambda i: (0, 0)),
        ],
        out_specs=[
            pl.BlockSpec((_BN, m), lambda i: (i, 0)),
            pl.BlockSpec((_BN, p), lambda i: (i, 0)),
        ],
        out_shape=[
            jax.ShapeDtypeStruct((n, m), jnp.float32),
            jax.ShapeDtypeStruct((n, p), jnp.float32),
        ],
    )(h, w, amat)


def _attn_mat(a_src, a_dst):
    H, C = a_src.shape
    hid = H * C
    m = np.zeros((hid, 2 * H), dtype=np.float32)
    blk = np.zeros((hid, 2 * H), dtype=np.float32)
    rows = np.arange(hid)
    heads = rows // C
    m[rows, heads] = 1.0
    blk[rows, H + heads] = 1.0
    a_s_flat = jnp.reshape(a_src, (hid,))
    a_d_flat = jnp.reshape(a_dst, (hid,))
    return jnp.asarray(m) * a_s_flat[:, None] + jnp.asarray(blk) * a_d_flat[:, None]


def kernel(x, edge_index, W_gcn, b_gcn, W_gat0, a_src0, a_dst0, b_gat0,
           W_gat1, a_src1, a_dst1, b_gat1, W_gat2, a_src2, a_dst2, b_gat2,
           W_out, b_out):
    n = x.shape[0]
    H, C = a_src0.shape
    src = jnp.asarray(edge_index[0], jnp.int32)
    dst = jnp.asarray(edge_index[1], jnp.int32)
    loop = jnp.arange(n, dtype=jnp.int32)
    s = jnp.concatenate([src, loop])
    d = jnp.concatenate([dst, loop])

    hlin = _mm(x, W_gcn)
    deg = jax.ops.segment_sum(jnp.ones_like(d, dtype=jnp.float32), d,
                              num_segments=n)
    dinv = 1.0 / jnp.sqrt(jnp.clip(deg, 1.0))
    norm = dinv[s] * dinv[d]
    h = jax.ops.segment_sum(norm[:, None] * hlin[s], d, num_segments=n)
    h = jax.nn.relu(h + b_gcn)

    gats = [(W_gat0, a_src0, a_dst0, b_gat0),
            (W_gat1, a_src1, a_dst1, b_gat1),
            (W_gat2, a_src2, a_dst2, b_gat2)]
    for (Wg, a_s, a_d, bg) in gats:
        hw, al = _gatpre(h, Wg, _attn_mat(a_s, a_d))
        al_s = al[:, :H]
        al_d = al[:, H:]
        e = jax.nn.leaky_relu(al_s[s] + al_d[d], negative_slope=0.2)
        ee = jnp.exp(jnp.clip(e, -30.0, 30.0) - 30.0)
        hh = hw.reshape(n, H, C)
        msg = (ee[:, :, None] * hh[s]).reshape(-1, H * C)
        acc = jax.ops.segment_sum(jnp.concatenate([ee, msg], axis=1), d,
                                  num_segments=n)
        denom = jnp.clip(acc[:, :H], 1e-16)
        num = acc[:, H:].reshape(n, H, C)
        h = (num / denom[:, :, None]).reshape(n, H * C) + bg

    return _mm(h, W_out) + b_out


# speedup vs baseline: 9.6512x; 2.2186x over previous
import jax
import jax.numpy as jnp
import numpy as np
from jax.experimental import pallas as pl

_N = 50000
_BN = 1000


def _mm_body(x_ref, w_ref, o_ref):
    o_ref[...] = jnp.dot(x_ref[...], w_ref[...],
                         preferred_element_type=jnp.float32)


def _mm(x, w):
    n, k = x.shape
    m = w.shape[1]
    return pl.pallas_call(
        _mm_body,
        grid=(n // _BN,),
        in_specs=[
            pl.BlockSpec((_BN, k), lambda i: (i, 0)),
            pl.BlockSpec((k, m), lambda i: (0, 0)),
        ],
        out_specs=pl.BlockSpec((_BN, m), lambda i: (i, 0)),
        out_shape=jax.ShapeDtypeStruct((n, m), jnp.float32),
    )(x, w)


def _gatpre_body(h_ref, w_ref, a_ref, hw_ref, al_ref):
    hw = jnp.dot(h_ref[...], w_ref[...], preferred_element_type=jnp.float32)
    hw_ref[...] = hw
    al_ref[...] = jnp.dot(hw, a_ref[...], preferred_element_type=jnp.float32)


def _gatpre(h, w, amat):
    n, k = h.shape
    m = w.shape[1]
    p = amat.shape[1]
    return pl.pallas_call(
        _gatpre_body,
        grid=(n // _BN,),
        in_specs=[
            pl.BlockSpec((_BN, k), lambda i: (i, 0)),
            pl.BlockSpec((k, m), lambda i: (0, 0)),
            pl.BlockSpec((m, p), lambda i: (0, 0)),
        ],
        out_specs=[
            pl.BlockSpec((_BN, m), lambda i: (i, 0)),
            pl.BlockSpec((_BN, p), lambda i: (i, 0)),
        ],
        out_shape=[
            jax.ShapeDtypeStruct((n, m), jnp.float32),
            jax.ShapeDtypeStruct((n, p), jnp.float32),
        ],
    )(h, w, amat)


_EB = 1000


def _edge_body(e_ref, hs_ref, r_ref, o_ref):
    e = e_ref[...]
    e = jnp.where(e > 0, e, 0.2 * e)
    ee = jnp.exp(jnp.clip(e, -30.0, 30.0) - 30.0)
    ee_wide = jnp.dot(ee, r_ref[...], preferred_element_type=jnp.float32)
    o_ref[...] = jnp.concatenate([ee, ee_wide * hs_ref[...]], axis=1)


def _edge_payload(e_pre, hs, H, C):
    E = e_pre.shape[0]
    hid = H * C
    rmat = jnp.asarray(np.repeat(np.eye(H, dtype=np.float32), C, axis=1))
    return pl.pallas_call(
        _edge_body,
        grid=(E // _EB,),
        in_specs=[
            pl.BlockSpec((_EB, H), lambda i: (i, 0)),
            pl.BlockSpec((_EB, hid), lambda i: (i, 0)),
            pl.BlockSpec((H, hid), lambda i: (0, 0)),
        ],
        out_specs=pl.BlockSpec((_EB, H + hid), lambda i: (i, 0)),
        out_shape=jax.ShapeDtypeStruct((E, H + hid), jnp.float32),
    )(e_pre, hs, rmat)


def _attn_mat(a_src, a_dst):
    H, C = a_src.shape
    hid = H * C
    m = np.zeros((hid, 2 * H), dtype=np.float32)
    blk = np.zeros((hid, 2 * H), dtype=np.float32)
    rows = np.arange(hid)
    heads = rows // C
    m[rows, heads] = 1.0
    blk[rows, H + heads] = 1.0
    a_s_flat = jnp.reshape(a_src, (hid,))
    a_d_flat = jnp.reshape(a_dst, (hid,))
    return jnp.asarray(m) * a_s_flat[:, None] + jnp.asarray(blk) * a_d_flat[:, None]


def kernel(x, edge_index, W_gcn, b_gcn, W_gat0, a_src0, a_dst0, b_gat0,
           W_gat1, a_src1, a_dst1, b_gat1, W_gat2, a_src2, a_dst2, b_gat2,
           W_out, b_out):
    n = x.shape[0]
    H, C = a_src0.shape
    src = jnp.asarray(edge_index[0], jnp.int32)
    dst = jnp.asarray(edge_index[1], jnp.int32)

    hlin = _mm(x, W_gcn)
    deg = jax.ops.segment_sum(jnp.ones_like(dst, dtype=jnp.float32), dst,
                              num_segments=n) + 1.0
    dinv = 1.0 / jnp.sqrt(jnp.clip(deg, 1.0))
    norm = dinv[src] * dinv[dst]
    h = jax.ops.segment_sum(norm[:, None] * hlin[src], dst, num_segments=n)
    h = h + (dinv * dinv)[:, None] * hlin
    h = jax.nn.relu(h + b_gcn)

    gats = [(W_gat0, a_src0, a_dst0, b_gat0),
            (W_gat1, a_src1, a_dst1, b_gat1),
            (W_gat2, a_src2, a_dst2, b_gat2)]
    for (Wg, a_s, a_d, bg) in gats:
        hw, al = _gatpre(h, Wg, _attn_mat(a_s, a_d))
        al_s = al[:, :H]
        al_d = al[:, H:]
        payload = _edge_payload(al_s[src] + al_d[dst], hw[src], H, C)
        acc = jax.ops.segment_sum(payload, dst, num_segments=n)
        e_self = al_s + al_d
        e_self = jnp.where(e_self > 0, e_self, 0.2 * e_self)
        ee_self = jnp.exp(jnp.clip(e_self, -30.0, 30.0) - 30.0)
        hh = hw.reshape(n, H, C)
        denom = jnp.clip(acc[:, :H] + ee_self, 1e-16)
        num = acc[:, H:].reshape(n, H, C) + ee_self[:, :, None] * hh
        h = (num / denom[:, :, None]).reshape(n, H * C) + bg

    return _mm(h, W_out) + b_out
